# baseline (speedup 1.0000x reference)
import functools

import jax
import jax.numpy as jnp
from jax import lax
from jax.experimental import pallas as pl
from jax.experimental.pallas import tpu as pltpu
from jax.experimental.pallas import tpu_sc as plsc

NUM_FIELDS = 26
VOCAB = 100000
HIDDEN = 128
BATCH = 4096

NC = 2
NS = 16
L = 16
NW = NC * NS
BPW = BATCH // NW
HS = HIDDEN // L


def _build_sc_kernel():
    mesh = plsc.VectorSubcoreMesh(core_axis_name="c", subcore_axis_name="s")

    @functools.partial(
        pl.kernel,
        mesh=mesh,
        out_type=jax.ShapeDtypeStruct((BATCH, HIDDEN), jnp.float32),
        scratch_types=[
            pltpu.VMEM((NUM_FIELDS, BPW), jnp.int32),
            pltpu.VMEM((2, BPW, HIDDEN), jnp.float32),
            pltpu.SemaphoreType.DMA,
            pltpu.SemaphoreType.DMA,
        ],
    )
    def k(xt_hbm, tbl_hbm, out_hbm, idx_v, acc_v, semi, sema):
        wid = lax.axis_index("s") * NC + lax.axis_index("c")
        base = wid * BPW

        stage = pltpu.async_copy(
            xt_hbm.at[:, pl.ds(base, BPW)], idx_v, semi)

        zero = jnp.zeros((L,), jnp.float32)

        def zrow(r, carry):
            for a in range(2):
                for h in range(HS):
                    acc_v[a, r, pl.ds(h * L, L)] = zero
            return carry
        lax.fori_loop(0, BPW, zrow, 0)

        stage.wait()

        descs = []
        for f in range(NUM_FIELDS):
            off = jnp.int32(f * VOCAB)
            for g in range(BPW // L):
                sl = pl.ds(g * L, L)
                idx_v[f, sl] = idx_v[f, sl] + off
            descs.append(
                pltpu.async_copy(tbl_hbm.at[idx_v.at[f]], acc_v.at[f % 2],
                                 sema, add=True))
        for d in descs:
            d.wait()

        def mrow(r, carry):
            for h in range(HS):
                sl = pl.ds(h * L, L)
                acc_v[0, r, sl] = acc_v[0, r, sl] + acc_v[1, r, sl]
            return carry
        lax.fori_loop(0, BPW, mrow, 0)

        pltpu.sync_copy(acc_v.at[0], out_hbm.at[pl.ds(base, BPW)])

    return k


_sc_call = _build_sc_kernel()


def kernel(x, tables):
    xt = x.T.reshape(NUM_FIELDS, BATCH).astype(jnp.int32)
    tbl = tables.reshape(NUM_FIELDS * VOCAB, HIDDEN)
    return _sc_call(xt, tbl)


# speedup vs baseline: 1.0615x; 1.0615x over previous
import functools

import jax
import jax.numpy as jnp
from jax import lax
from jax.experimental import pallas as pl
from jax.experimental.pallas import tpu as pltpu
from jax.experimental.pallas import tpu_sc as plsc

NUM_FIELDS = 26
VOCAB = 100000
HIDDEN = 128
BATCH = 4096

NC = 2
NS = 16
L = 16
NW = NC * NS
BPW = BATCH // NW
HS = HIDDEN // L


def _build_sc_kernel():
    mesh = plsc.VectorSubcoreMesh(core_axis_name="c", subcore_axis_name="s")

    @functools.partial(
        pl.kernel,
        mesh=mesh,
        out_type=jax.ShapeDtypeStruct((BATCH, HIDDEN), jnp.float32),
        scratch_types=[
            pltpu.VMEM((NUM_FIELDS, BPW), jnp.int32),
            pltpu.VMEM((BPW, HIDDEN), jnp.float32),
            pltpu.SemaphoreType.DMA,
            pltpu.SemaphoreType.DMA,
        ],
    )
    def k(xt_hbm, tbl_hbm, out_hbm, idx_v, acc_v, semi, sema):
        wid = lax.axis_index("s") * NC + lax.axis_index("c")
        base = wid * BPW

        stage = pltpu.async_copy(
            xt_hbm.at[:, pl.ds(base, BPW)], idx_v, semi)

        zero = jnp.zeros((L,), jnp.float32)

        def zrow(r, carry):
            for h in range(HS):
                acc_v[r, pl.ds(h * L, L)] = zero
            return carry
        lax.fori_loop(0, BPW, zrow, 0)

        stage.wait()

        descs = []
        for f in range(NUM_FIELDS):
            descs.append(
                pltpu.async_copy(tbl_hbm.at[f].at[idx_v.at[f]], acc_v, sema,
                                 add=True))
        for d in descs:
            d.wait()

        pltpu.sync_copy(acc_v, out_hbm.at[pl.ds(base, BPW)])

    return k


_sc_call = _build_sc_kernel()


def kernel(x, tables):
    xt = x.T.reshape(NUM_FIELDS, BATCH).astype(jnp.int32)
    return _sc_call(xt, tables)
